# baseline (speedup 1.0000x reference)
import functools

import jax
import jax.numpy as jnp
from jax import lax
from jax.experimental import pallas as pl
from jax.experimental.pallas import tpu as pltpu
from jax.experimental.pallas import tpu_sc as plsc

B = 4
P = 1024
K = 30
N = B * P
E = N * K
BIG = 1e30


def _pad_cols(a, m=8):
    d = a.shape[-1]
    pad = (-d) % m
    if pad == 0:
        return a
    return jnp.concatenate([a, jnp.zeros(a.shape[:-1] + (pad,), a.dtype)], axis=-1)


def _pad_rows(a, m=8):
    d = a.shape[0]
    pad = (-d) % m
    if pad == 0:
        return a
    return jnp.concatenate([a, jnp.zeros((pad,) + a.shape[1:], a.dtype)], axis=0)


def _knn_body(f_ref, o_ref):
    b = pl.program_id(0)
    fb = f_ref[0]
    sq = jnp.sum(fb * fb, axis=1, keepdims=True)
    cross = lax.dot_general(fb, fb, (((1,), (1,)), ((), ())),
                            preferred_element_type=jnp.float32)
    d2 = sq + sq.T - 2.0 * cross
    col = lax.broadcasted_iota(jnp.int32, (P, P), 1)
    row = lax.broadcasted_iota(jnp.int32, (P, P), 0)
    d2 = jnp.where(col == row, 1e10, d2)
    colf = col.astype(jnp.float32)
    rows = []
    for _ in range(K):
        m = jnp.min(d2, axis=1, keepdims=True)
        eq = d2 <= m
        amin = jnp.min(jnp.where(eq, colf, 1e9), axis=1)
        rows.append(amin[None, :])
        d2 = jnp.where(eq, BIG, d2)
    rows.append(jnp.zeros((2, P), jnp.float32))
    nbr = jnp.concatenate(rows, axis=0).astype(jnp.int32)
    o_ref[0] = nbr + b * P


def _knn(f):
    D = f.shape[-1]
    return pl.pallas_call(
        _knn_body,
        grid=(B,),
        in_specs=[pl.BlockSpec((1, P, D), lambda b: (b, 0, 0))],
        out_specs=pl.BlockSpec((1, 32, P), lambda b: (b, 0, 0)),
        out_shape=jax.ShapeDtypeStruct((B, 32, P), jnp.int32),
    )(f)


def _matmul_parts(parts, ws, bias):
    n = parts[0].shape[0]
    dout = ws[0].shape[1]
    np_ = len(parts)
    blk = 512

    def body(*refs):
        o_ref = refs[-1]
        acc = jnp.dot(refs[0][...], refs[np_][...],
                      preferred_element_type=jnp.float32, precision=lax.Precision.HIGHEST)
        for i in range(1, np_):
            acc += jnp.dot(refs[i][...], refs[np_ + i][...],
                           preferred_element_type=jnp.float32, precision=lax.Precision.HIGHEST)
        o_ref[...] = acc + refs[2 * np_][...]

    in_specs = [
        pl.BlockSpec((blk, p.shape[1]), (lambda j: (j, 0))) for p in parts
    ] + [
        pl.BlockSpec(wv.shape, (lambda j: (0, 0))) for wv in ws
    ] + [pl.BlockSpec((1, dout), lambda j: (0, 0))]
    return pl.pallas_call(
        body,
        grid=(n // blk,),
        in_specs=in_specs,
        out_specs=pl.BlockSpec((blk, dout), lambda j: (j, 0)),
        out_shape=jax.ShapeDtypeStruct((n, dout), jnp.float32),
    )(*parts, *ws, bias)


def _row_splits(w, dims):
    out = []
    off = 0
    for d in dims:
        wi = w[off:off + d]
        out.append(_pad_rows(wi) if d % 8 else wi)
        off += d
    return out


def _sc_gather(table, idx2d, F):
    nw = 32
    rpw = idx2d.shape[1]

    mesh = plsc.VectorSubcoreMesh(core_axis_name="c", subcore_axis_name="s")

    @functools.partial(
        pl.kernel,
        mesh=mesh,
        compiler_params=pltpu.CompilerParams(use_tc_tiling_on_sc=False),
        out_type=jax.ShapeDtypeStruct((E, F), jnp.float32),
        scratch_types=[
            pltpu.VMEM((rpw, 128), jnp.int32),
            pltpu.VMEM((128, F), jnp.float32),
            pltpu.VMEM((128, F), jnp.float32),
            pltpu.SemaphoreType.DMA,
            pltpu.SemaphoreType.DMA,
            pltpu.SemaphoreType.DMA,
            pltpu.SemaphoreType.DMA,
        ],
    )
    def gk(idx_hbm, table_hbm, out_hbm, idx_v, rows_a, rows_b, sia, sib,
           soa, sob):
        c = lax.axis_index("c")
        s = lax.axis_index("s")
        wid = s * 2 + c
        pltpu.sync_copy(idx_hbm.at[wid], idx_v)

        rows = (rows_a, rows_b)
        sin = (sia, sib)
        sout = (soa, sob)

        def start_in(j, b):
            return pltpu.async_copy(table_hbm.at[idx_v.at[j]], rows[b], sin[b])

        def start_out(j, b):
            return pltpu.async_copy(
                rows[b], out_hbm.at[pl.ds((wid * rpw + j) * 128, 128)],
                sout[b])

        hin = [None] * rpw
        hout = [None] * rpw
        hin[0] = start_in(0, 0)
        for j in range(rpw):
            b = j % 2
            if j + 1 < rpw:
                if j >= 1:
                    hout[j - 1].wait()
                hin[j + 1] = start_in(j + 1, (j + 1) % 2)
            hin[j].wait()
            hout[j] = start_out(j, b)
        if rpw >= 2:
            hout[rpw - 2].wait()
        hout[rpw - 1].wait()

    return gk(idx2d, table)


def _sc_gather_multi(tables, idx2d):
    nt = len(tables)
    rpw = idx2d.shape[1]

    mesh = plsc.VectorSubcoreMesh(core_axis_name="c", subcore_axis_name="s")
    scratch = [pltpu.VMEM((rpw, 128), jnp.int32)]
    for t in tables:
        scratch += [pltpu.VMEM((128, t.shape[1]), jnp.float32),
                    pltpu.VMEM((128, t.shape[1]), jnp.float32)]
    scratch += [pltpu.SemaphoreType.DMA] * (4 * nt)

    @functools.partial(
        pl.kernel,
        mesh=mesh,
        compiler_params=pltpu.CompilerParams(use_tc_tiling_on_sc=False),
        out_type=[jax.ShapeDtypeStruct((E, t.shape[1]), jnp.float32)
                  for t in tables],
        scratch_types=scratch,
    )
    def gk(idx_hbm, *refs):
        tabs = refs[:nt]
        outs = refs[nt:2 * nt]
        idx_v = refs[2 * nt]
        bufs = refs[2 * nt + 1:2 * nt + 1 + 2 * nt]
        sems = refs[2 * nt + 1 + 2 * nt:]
        c = lax.axis_index("c")
        s = lax.axis_index("s")
        wid = s * 2 + c
        pltpu.sync_copy(idx_hbm.at[wid], idx_v)

        def start_in(j, b):
            return [pltpu.async_copy(tabs[t].at[idx_v.at[j]],
                                     bufs[2 * t + b], sems[4 * t + b])
                    for t in range(nt)]

        def start_out(j, b):
            return [pltpu.async_copy(
                bufs[2 * t + b],
                outs[t].at[pl.ds((wid * rpw + j) * 128, 128)],
                sems[4 * t + 2 + b]) for t in range(nt)]

        hin = [None] * rpw
        hout = [None] * rpw
        hin[0] = start_in(0, 0)
        for j in range(rpw):
            b = j % 2
            if j + 1 < rpw:
                if j >= 1:
                    for h in hout[j - 1]:
                        h.wait()
                hin[j + 1] = start_in(j + 1, (j + 1) % 2)
            for h in hin[j]:
                h.wait()
            hout[j] = start_out(j, b)
        for jj in ([rpw - 2] if rpw >= 2 else []) + [rpw - 1]:
            for h in hout[jj]:
                h.wait()

    outs = gk(idx2d, *tables)
    return list(outs) if isinstance(outs, (list, tuple)) else [outs]


def _edge_body(w_ref, vg_ref, Wg_ref, W2_ref, b2_ref, g_ref, bt_ref, o_ref,
               st_ref):
    p = pl.program_id(0)
    j = pl.program_id(1)
    kk, r, dg = vg_ref.shape
    f1 = w_ref.shape[1]
    vgm = jnp.reshape(vg_ref[...], (kk * r, dg))
    if Wg_ref is not None:
        vgm = jnp.dot(vgm, Wg_ref[...], preferred_element_type=jnp.float32, precision=lax.Precision.HIGHEST)
    h1 = jnp.reshape(vgm, (kk, r, f1)) + w_ref[...][None]

    @pl.when((p == 0) & (j == 0))
    def _():
        st_ref[...] = jnp.zeros_like(st_ref)

    @pl.when(p == 0)
    def _():
        s1 = jnp.sum(h1, axis=0)
        s2 = jnp.sum(h1 * h1, axis=0)
        s1 = jnp.sum(jnp.reshape(s1, (r // 8, 8, f1)), axis=0)
        s2 = jnp.sum(jnp.reshape(s2, (r // 8, 8, f1)), axis=0)
        st_ref[...] += jnp.concatenate([s1, s2], axis=1)
        o_ref[...] = jnp.zeros_like(o_ref)

    @pl.when(p == 1)
    def _():
        cnt = float(E)
        tot = jnp.sum(st_ref[...], axis=0, keepdims=True)
        mu = tot[:, 0:f1] / cnt
        var = tot[:, f1:] / cnt - mu * mu
        sc = g_ref[...] * (1.0 / jnp.sqrt(var + 1e-5))
        hn = (h1 - mu[None]) * sc[None] + bt_ref[...][None]
        hr = jnp.maximum(hn, 0.0)
        h2 = jnp.dot(
            jnp.reshape(hr, (kk * r, f1)),
            W2_ref[...],
            preferred_element_type=jnp.float32, precision=lax.Precision.HIGHEST,
        ) + b2_ref[...]
        f2 = h2.shape[1]
        o_ref[...] = jnp.max(jnp.reshape(h2, (kk, r, f2)), axis=0)


def _edge_stage(w, vg3, W2, b2, gamma, beta, Wg=None):
    F1 = w.shape[1]
    F2 = W2.shape[1]
    Dg = vg3.shape[2]
    nblk = 16
    R = N // nblk
    body = _edge_body if Wg is not None else (
        lambda w_r, vg_r, W2_r, b2_r, g_r, bt_r, o_r, st_r:
        _edge_body(w_r, vg_r, None, W2_r, b2_r, g_r, bt_r, o_r, st_r))
    in_specs = [
        pl.BlockSpec((R, F1), lambda p, j: (j, 0)),
        pl.BlockSpec((K, R, Dg), lambda p, j: (0, j, 0)),
    ]
    args = [w, vg3]
    if Wg is not None:
        in_specs.append(pl.BlockSpec((Dg, F1), lambda p, j: (0, 0)))
        args.append(Wg)
    in_specs += [
        pl.BlockSpec((F1, F2), lambda p, j: (0, 0)),
        pl.BlockSpec((1, F2), lambda p, j: (0, 0)),
        pl.BlockSpec((1, F1), lambda p, j: (0, 0)),
        pl.BlockSpec((1, F1), lambda p, j: (0, 0)),
    ]
    args += [W2, b2[None], gamma[None], beta[None]]
    return pl.pallas_call(
        body,
        grid=(2, nblk),
        in_specs=in_specs,
        out_specs=pl.BlockSpec((R, F2), lambda p, j: (j, 0)),
        out_shape=jax.ShapeDtypeStruct((N, F2), jnp.float32),
        scratch_shapes=[pltpu.VMEM((8, 2 * F1), jnp.float32)],
        compiler_params=pltpu.CompilerParams(
            vmem_limit_bytes=100 * 1024 * 1024),
    )(*args)


def _edge_conv_mimic(parts, dims, nbr, layers):
    (W1, b1, gamma, beta), (W2, b2) = layers
    np_ = len(parts)
    F1 = W2.shape[0]
    d = sum(dims)
    idx = jnp.reshape(jnp.transpose(nbr[:, :K, :], (1, 0, 2)),
                      (32, E // (32 * 128), 128))
    tables = [_pad_cols(pt, 16) for pt in parts]
    xgs = _sc_gather_multi(tables, idx)
    xg3 = [jnp.reshape(xg, (K, N, t.shape[1]))
           for xg, t in zip(xgs, tables)]
    Wts = _row_splits(W1[:d], dims)
    Wbs = _row_splits(W1[d:], dims)
    nblk = 16
    R = N // nblk

    def body(*refs):
        xi = refs[:np_]
        xg = refs[np_:2 * np_]
        wt = refs[2 * np_:3 * np_]
        wb = refs[3 * np_:4 * np_]
        b1_r, W2_r, b2_r, g_r, bt_r = refs[4 * np_:4 * np_ + 5]
        o_ref, st_ref = refs[-2:]
        p = pl.program_id(0)
        j = pl.program_id(1)
        kk = K
        f1 = F1

        u = None
        for t in range(np_):
            dp = xi[t].shape[1]
            ut = jnp.dot(xi[t][...], wt[t][...],
                         preferred_element_type=jnp.float32)
            u = ut if u is None else u + ut
            diff = xg[t][...][:, :, 0:dp] - xi[t][...][None]
            vt = jnp.dot(jnp.reshape(diff, (kk * R, dp)), wb[t][...],
                         preferred_element_type=jnp.float32)
            vt = jnp.reshape(vt, (kk, R, f1))
            u2 = vt if t == 0 else u2 + vt
        h1 = u[None] + u2 + b1_r[...][None]

        @pl.when((p == 0) & (j == 0))
        def _():
            st_ref[...] = jnp.zeros_like(st_ref)

        @pl.when(p == 0)
        def _():
            s1 = jnp.sum(h1, axis=0)
            s2 = jnp.sum(h1 * h1, axis=0)
            s1 = jnp.sum(jnp.reshape(s1, (R // 8, 8, f1)), axis=0)
            s2 = jnp.sum(jnp.reshape(s2, (R // 8, 8, f1)), axis=0)
            st_ref[...] += jnp.concatenate([s1, s2], axis=1)
            o_ref[...] = jnp.zeros_like(o_ref)

        @pl.when(p == 1)
        def _():
            cnt = float(E)
            tot = jnp.sum(st_ref[...], axis=0, keepdims=True)
            mu = tot[:, 0:f1] / cnt
            var = tot[:, f1:] / cnt - mu * mu
            sq = jnp.sqrt(var + 1e-5)
            hn = (h1 - mu[None]) / sq[None] * g_r[...][None] \
                + bt_r[...][None]
            hr = jnp.maximum(hn, 0.0)
            h2 = jnp.dot(jnp.reshape(hr, (kk * R, f1)), W2_r[...],
                         preferred_element_type=jnp.float32) + b2_r[...]
            f2 = h2.shape[1]
            o_ref[...] = jnp.max(jnp.reshape(h2, (kk, R, f2)), axis=0)

    cst = lambda shape: pl.BlockSpec(shape, lambda p, j: tuple(0 for _ in shape))
    in_specs = (
        [pl.BlockSpec((R, pt.shape[1]), (lambda p, j: (j, 0)))
         for pt in parts]
        + [pl.BlockSpec((K, R, x3.shape[2]), (lambda p, j: (0, j, 0)))
           for x3 in xg3]
        + [cst(wv.shape) for wv in Wts]
        + [cst(wv.shape) for wv in Wbs]
        + [cst((1, F1)), cst((F1, W2.shape[1])), cst((1, W2.shape[1])),
           cst((1, F1)), cst((1, F1))]
    )
    return pl.pallas_call(
        body,
        grid=(2, nblk),
        in_specs=in_specs,
        out_specs=pl.BlockSpec((R, W2.shape[1]), lambda p, j: (j, 0)),
        out_shape=jax.ShapeDtypeStruct((N, W2.shape[1]), jnp.float32),
        scratch_shapes=[pltpu.VMEM((8, 2 * F1), jnp.float32)],
        compiler_params=pltpu.CompilerParams(
            vmem_limit_bytes=100 * 1024 * 1024),
    )(*parts, *xg3, *Wts, *Wbs, b1[None], W2, b2[None], gamma[None],
      beta[None])


def _edge_conv(parts, dims, nbr, layers, raw_gather=False):
    (W1, b1, gamma, beta), (W2, b2) = layers
    d = sum(dims)
    Wt, Wb = W1[:d], W1[d:]
    F1 = W2.shape[0]
    idx = jnp.reshape(jnp.transpose(nbr[:, :K, :], (1, 0, 2)),
                      (32, E // (32 * 128), 128))
    if raw_gather:
        w = _matmul_parts(parts, _row_splits(Wt - Wb, dims), b1[None])
        table = _pad_cols(parts[0], 16) if len(parts) == 1 else None
        dg = table.shape[1]
        vg = _sc_gather(table, idx, dg)
        vg3 = jnp.reshape(vg, (K, N, dg))
        return _edge_stage(w, vg3, W2, b2, gamma, beta,
                           Wg=_pad_rows(Wb, 16))
    Wd = jnp.concatenate([Wt - Wb, Wb], axis=1)
    bias = jnp.concatenate([b1, jnp.zeros_like(b1)])[None]
    uv = _matmul_parts(parts, _row_splits(Wd, dims), bias)
    w = uv[:, :F1]
    v = uv[:, F1:]
    vg = _sc_gather(v, idx, F1)
    vg3 = jnp.reshape(vg, (K, N, F1))
    return _edge_stage(w, vg3, W2, b2, gamma, beta)


def _fe_head(parts, dims, fe, head):
    (W1, b1, gamma, beta), (W2, b2) = fe
    (W0, b0), (W1h, b1h), (W2h, b2h), (W3, b3) = head
    H = W1.shape[1]
    d = sum(dims)
    np_ = len(parts)
    W1s = _row_splits(W1, dims)
    W0ts = _row_splits(W0[:d], dims)
    W0b = W0[d:]
    W3p = _pad_cols(W3, 64)
    b3p = _pad_cols(b3[None], 64)
    nblk = 8
    R = N // nblk

    def body(*refs):
        (o_ref, st_ref, mx_ref) = refs[-3:]
        prefs = refs[:np_]
        w1refs = refs[np_:2 * np_]
        (b1_r, g_r, bt_r, W2_r, b2_r) = refs[2 * np_:2 * np_ + 5]
        w0refs = refs[2 * np_ + 5:3 * np_ + 5]
        (W0b_r, b0_r, W1h_r, b1h_r, W2h_r, b2h_r, W3_r, b3_r) = \
            refs[3 * np_ + 5:3 * np_ + 13]
        p = pl.program_id(0)
        j = pl.program_id(1)

        def hidden():
            t = jnp.dot(prefs[0][...], w1refs[0][...],
                        preferred_element_type=jnp.float32, precision=lax.Precision.HIGHEST)
            for i in range(1, np_):
                t += jnp.dot(prefs[i][...], w1refs[i][...],
                             preferred_element_type=jnp.float32, precision=lax.Precision.HIGHEST)
            return t + b1_r[...]

        @pl.when((p == 0) & (j == 0))
        def _():
            st_ref[...] = jnp.zeros_like(st_ref)

        @pl.when(p == 0)
        def _():
            t = hidden()
            s1 = jnp.sum(jnp.reshape(t, (R // 8, 8, H)), axis=0)
            s2 = jnp.sum(jnp.reshape(t * t, (R // 8, 8, H)), axis=0)
            st_ref[...] += jnp.concatenate([s1, s2], axis=1)
            o_ref[...] = jnp.zeros_like(o_ref)

        @pl.when(p == 1)
        def _():
            t = hidden()
            cnt = float(N)
            tot = jnp.sum(st_ref[...], axis=0, keepdims=True)
            mu = tot[:, 0:H] / cnt
            var = tot[:, H:] / cnt - mu * mu
            sc = g_r[...] * (1.0 / jnp.sqrt(var + 1e-5))
            hr = jnp.maximum((t - mu) * sc + bt_r[...], 0.0)
            x5 = jnp.dot(hr, W2_r[...],
                         preferred_element_type=jnp.float32, precision=lax.Precision.HIGHEST) + b2_r[...]
            bm = jnp.max(x5, axis=0)[None]

            @pl.when(j == 0)
            def _():
                mx_ref[...] = jnp.full_like(mx_ref, -BIG)

            mx_ref[...] = jnp.maximum(mx_ref[...], bm)
            o_ref[...] = jnp.zeros_like(o_ref)

        @pl.when(p == 2)
        def _():
            gbias = jnp.dot(mx_ref[...], W0b_r[...],
                            preferred_element_type=jnp.float32, precision=lax.Precision.HIGHEST) + b0_r[...]
            h = jnp.dot(prefs[0][...], w0refs[0][...],
                        preferred_element_type=jnp.float32, precision=lax.Precision.HIGHEST)
            for i in range(1, np_):
                h += jnp.dot(prefs[i][...], w0refs[i][...],
                             preferred_element_type=jnp.float32, precision=lax.Precision.HIGHEST)
            h = jnp.maximum(h + gbias, 0.0)
            h = jnp.maximum(
                jnp.dot(h, W1h_r[...], preferred_element_type=jnp.float32, precision=lax.Precision.HIGHEST)
                + b1h_r[...], 0.0)
            h = jnp.maximum(
                jnp.dot(h, W2h_r[...], preferred_element_type=jnp.float32, precision=lax.Precision.HIGHEST)
                + b2h_r[...], 0.0)
            o = jnp.dot(h, W3_r[...], preferred_element_type=jnp.float32, precision=lax.Precision.HIGHEST) \
                + b3_r[...]
            colmask = lax.broadcasted_iota(jnp.int32, o.shape, 1) < 50
            o = jnp.where(colmask, o, -BIG)
            m = jnp.max(o, axis=1, keepdims=True)
            z = jnp.sum(jnp.exp(o - m), axis=1, keepdims=True)
            o_ref[...] = o - m - jnp.log(z)

    cst = lambda shape: pl.BlockSpec(shape, lambda p, j: tuple(0 for _ in shape))
    in_specs = (
        [pl.BlockSpec((R, pt.shape[1]), (lambda p, j: (j, 0))) for pt in parts]
        + [cst(wv.shape) for wv in W1s]
        + [cst((1, H)), cst((1, H)), cst((1, H)), cst((H, H)), cst((1, H))]
        + [cst(wv.shape) for wv in W0ts]
        + [cst((H, 256)), cst((1, 256)), cst((256, 256)), cst((1, 256)),
           cst((256, 128)), cst((1, 128)), cst((128, 64)), cst((1, 64))]
    )
    out = pl.pallas_call(
        body,
        grid=(3, nblk),
        in_specs=in_specs,
        out_specs=pl.BlockSpec((R, 64), lambda p, j: (j, 0)),
        out_shape=jax.ShapeDtypeStruct((N, 64), jnp.float32),
        scratch_shapes=[
            pltpu.VMEM((8, 2 * H), jnp.float32),
            pltpu.VMEM((1, H), jnp.float32),
        ],
    )(*parts, *W1s, b1[None], gamma[None], beta[None], W2, b2[None],
      *W0ts, W0b, b0[None], W1h, b1h[None], W2h, b2h[None], W3p, b3p)
    return out[:, :50]


def kernel(x, pos, batch, params):
    x0p = _pad_cols(jnp.concatenate([x, pos], axis=1))

    nbr = _knn(x0p.reshape(B, P, 8))
    x1 = _edge_conv_mimic([x0p], [6], nbr, params["ec1"])

    nbr = _knn(x1.reshape(B, P, 64))
    x2 = _edge_conv_mimic([x0p, x1], [6, 64], nbr, params["ec2"])

    nbr = _knn(x2.reshape(B, P, 64))
    x3 = _edge_conv([x0p, x1, x2], [6, 64, 64], nbr, params["ec3"])

    x4 = _edge_conv([x0p, x1, x2, x3], [6, 64, 64, 64], nbr, params["ec4"])

    return _fe_head([x0p, x1, x2, x3, x4], [6, 64, 64, 64, 128],
                    params["fe"], params["head"])


# speedup vs baseline: 1.2630x; 1.2630x over previous
import functools

import jax
import jax.numpy as jnp
from jax import lax
from jax.experimental import pallas as pl
from jax.experimental.pallas import tpu as pltpu
from jax.experimental.pallas import tpu_sc as plsc

B = 4
P = 1024
K = 30
N = B * P
E = N * K
BIG = 1e30


def _pad_cols(a, m=8):
    d = a.shape[-1]
    pad = (-d) % m
    if pad == 0:
        return a
    return jnp.concatenate([a, jnp.zeros(a.shape[:-1] + (pad,), a.dtype)], axis=-1)


def _pad_rows(a, m=8):
    d = a.shape[0]
    pad = (-d) % m
    if pad == 0:
        return a
    return jnp.concatenate([a, jnp.zeros((pad,) + a.shape[1:], a.dtype)], axis=0)


def _knn_body(f_ref, o_ref):
    b = pl.program_id(0)
    fb = f_ref[0]
    sq = jnp.sum(fb * fb, axis=1, keepdims=True)
    cross = lax.dot_general(fb, fb, (((1,), (1,)), ((), ())),
                            preferred_element_type=jnp.float32)
    d2 = sq + sq.T - 2.0 * cross
    col = lax.broadcasted_iota(jnp.int32, (P, P), 1)
    row = lax.broadcasted_iota(jnp.int32, (P, P), 0)
    d2 = jnp.where(col == row, 1e10, d2)
    colf = col.astype(jnp.float32)
    rows = []
    for _ in range(K):
        m = jnp.min(d2, axis=1, keepdims=True)
        eq = d2 <= m
        amin = jnp.min(jnp.where(eq, colf, 1e9), axis=1)
        rows.append(amin[None, :])
        d2 = jnp.where(eq, BIG, d2)
    rows.append(jnp.zeros((2, P), jnp.float32))
    nbr = jnp.concatenate(rows, axis=0).astype(jnp.int32)
    o_ref[0] = nbr + b * P


def _knn(f):
    D = f.shape[-1]
    return pl.pallas_call(
        _knn_body,
        grid=(B,),
        in_specs=[pl.BlockSpec((1, P, D), lambda b: (b, 0, 0))],
        out_specs=pl.BlockSpec((1, 32, P), lambda b: (b, 0, 0)),
        out_shape=jax.ShapeDtypeStruct((B, 32, P), jnp.int32),
    )(f)


def _matmul_parts(parts, ws, bias):
    n = parts[0].shape[0]
    dout = ws[0].shape[1]
    np_ = len(parts)
    blk = 512

    def body(*refs):
        o_ref = refs[-1]
        acc = jnp.dot(refs[0][...], refs[np_][...],
                      preferred_element_type=jnp.float32)
        for i in range(1, np_):
            acc += jnp.dot(refs[i][...], refs[np_ + i][...],
                           preferred_element_type=jnp.float32)
        o_ref[...] = acc + refs[2 * np_][...]

    in_specs = [
        pl.BlockSpec((blk, p.shape[1]), (lambda j: (j, 0))) for p in parts
    ] + [
        pl.BlockSpec(wv.shape, (lambda j: (0, 0))) for wv in ws
    ] + [pl.BlockSpec((1, dout), lambda j: (0, 0))]
    return pl.pallas_call(
        body,
        grid=(n // blk,),
        in_specs=in_specs,
        out_specs=pl.BlockSpec((blk, dout), lambda j: (j, 0)),
        out_shape=jax.ShapeDtypeStruct((n, dout), jnp.float32),
    )(*parts, *ws, bias)


def _row_splits(w, dims):
    out = []
    off = 0
    for d in dims:
        wi = w[off:off + d]
        out.append(_pad_rows(wi) if d % 8 else wi)
        off += d
    return out


def _sc_gather(table, idx2d, F):
    nw = 32
    rpw = idx2d.shape[1]

    mesh = plsc.VectorSubcoreMesh(core_axis_name="c", subcore_axis_name="s")

    @functools.partial(
        pl.kernel,
        mesh=mesh,
        compiler_params=pltpu.CompilerParams(use_tc_tiling_on_sc=False),
        out_type=jax.ShapeDtypeStruct((E, F), jnp.float32),
        scratch_types=[
            pltpu.VMEM((rpw, 128), jnp.int32),
            pltpu.VMEM((128, F), jnp.float32),
            pltpu.VMEM((128, F), jnp.float32),
            pltpu.SemaphoreType.DMA,
            pltpu.SemaphoreType.DMA,
            pltpu.SemaphoreType.DMA,
            pltpu.SemaphoreType.DMA,
        ],
    )
    def gk(idx_hbm, table_hbm, out_hbm, idx_v, rows_a, rows_b, sia, sib,
           soa, sob):
        c = lax.axis_index("c")
        s = lax.axis_index("s")
        wid = s * 2 + c
        pltpu.sync_copy(idx_hbm.at[wid], idx_v)

        rows = (rows_a, rows_b)
        sin = (sia, sib)
        sout = (soa, sob)

        def start_in(j, b):
            return pltpu.async_copy(table_hbm.at[idx_v.at[j]], rows[b], sin[b])

        def start_out(j, b):
            return pltpu.async_copy(
                rows[b], out_hbm.at[pl.ds((wid * rpw + j) * 128, 128)],
                sout[b])

        hin = [None] * rpw
        hout = [None] * rpw
        hin[0] = start_in(0, 0)
        for j in range(rpw):
            b = j % 2
            if j + 1 < rpw:
                if j >= 1:
                    hout[j - 1].wait()
                hin[j + 1] = start_in(j + 1, (j + 1) % 2)
            hin[j].wait()
            hout[j] = start_out(j, b)
        if rpw >= 2:
            hout[rpw - 2].wait()
        hout[rpw - 1].wait()

    return gk(idx2d, table)


def _sc_gather_multi(tables, idx2d):
    nt = len(tables)
    rpw = idx2d.shape[1]

    mesh = plsc.VectorSubcoreMesh(core_axis_name="c", subcore_axis_name="s")
    scratch = [pltpu.VMEM((rpw, 128), jnp.int32)]
    for t in tables:
        scratch += [pltpu.VMEM((128, t.shape[1]), jnp.float32),
                    pltpu.VMEM((128, t.shape[1]), jnp.float32)]
    scratch += [pltpu.SemaphoreType.DMA] * (4 * nt)

    @functools.partial(
        pl.kernel,
        mesh=mesh,
        compiler_params=pltpu.CompilerParams(use_tc_tiling_on_sc=False),
        out_type=[jax.ShapeDtypeStruct((E, t.shape[1]), jnp.float32)
                  for t in tables],
        scratch_types=scratch,
    )
    def gk(idx_hbm, *refs):
        tabs = refs[:nt]
        outs = refs[nt:2 * nt]
        idx_v = refs[2 * nt]
        bufs = refs[2 * nt + 1:2 * nt + 1 + 2 * nt]
        sems = refs[2 * nt + 1 + 2 * nt:]
        c = lax.axis_index("c")
        s = lax.axis_index("s")
        wid = s * 2 + c
        pltpu.sync_copy(idx_hbm.at[wid], idx_v)

        def start_in(j, b):
            return [pltpu.async_copy(tabs[t].at[idx_v.at[j]],
                                     bufs[2 * t + b], sems[4 * t + b])
                    for t in range(nt)]

        def start_out(j, b):
            return [pltpu.async_copy(
                bufs[2 * t + b],
                outs[t].at[pl.ds((wid * rpw + j) * 128, 128)],
                sems[4 * t + 2 + b]) for t in range(nt)]

        hin = [None] * rpw
        hout = [None] * rpw
        hin[0] = start_in(0, 0)
        for j in range(rpw):
            b = j % 2
            if j + 1 < rpw:
                if j >= 1:
                    for h in hout[j - 1]:
                        h.wait()
                hin[j + 1] = start_in(j + 1, (j + 1) % 2)
            for h in hin[j]:
                h.wait()
            hout[j] = start_out(j, b)
        for jj in ([rpw - 2] if rpw >= 2 else []) + [rpw - 1]:
            for h in hout[jj]:
                h.wait()

    outs = gk(idx2d, *tables)
    return list(outs) if isinstance(outs, (list, tuple)) else [outs]


def _edge_body(w_ref, vg_ref, Wg_ref, W2_ref, b2_ref, g_ref, bt_ref, o_ref,
               st_ref):
    p = pl.program_id(0)
    j = pl.program_id(1)
    kk, r, dg = vg_ref.shape
    f1 = w_ref.shape[1]
    vgm = jnp.reshape(vg_ref[...], (kk * r, dg))
    if Wg_ref is not None:
        vgm = jnp.dot(vgm, Wg_ref[...], preferred_element_type=jnp.float32)
    h1 = jnp.reshape(vgm, (kk, r, f1)) + w_ref[...][None]

    @pl.when((p == 0) & (j == 0))
    def _():
        st_ref[...] = jnp.zeros_like(st_ref)

    @pl.when(p == 0)
    def _():
        s1 = jnp.sum(h1, axis=0)
        s2 = jnp.sum(h1 * h1, axis=0)
        s1 = jnp.sum(jnp.reshape(s1, (r // 8, 8, f1)), axis=0)
        s2 = jnp.sum(jnp.reshape(s2, (r // 8, 8, f1)), axis=0)
        st_ref[...] += jnp.concatenate([s1, s2], axis=1)
        o_ref[...] = jnp.zeros_like(o_ref)

    @pl.when(p == 1)
    def _():
        cnt = float(E)
        tot = jnp.sum(st_ref[...], axis=0, keepdims=True)
        mu = tot[:, 0:f1] / cnt
        var = tot[:, f1:] / cnt - mu * mu
        sc = g_ref[...] * (1.0 / jnp.sqrt(var + 1e-5))
        hn = (h1 - mu[None]) * sc[None] + bt_ref[...][None]
        hr = jnp.maximum(hn, 0.0)
        h2 = jnp.dot(
            jnp.reshape(hr, (kk * r, f1)),
            W2_ref[...],
            preferred_element_type=jnp.float32,
        ) + b2_ref[...]
        f2 = h2.shape[1]
        o_ref[...] = jnp.max(jnp.reshape(h2, (kk, r, f2)), axis=0)


def _edge_stage(w, vg3, W2, b2, gamma, beta, Wg=None):
    F1 = w.shape[1]
    F2 = W2.shape[1]
    Dg = vg3.shape[2]
    nblk = 16
    R = N // nblk
    body = _edge_body if Wg is not None else (
        lambda w_r, vg_r, W2_r, b2_r, g_r, bt_r, o_r, st_r:
        _edge_body(w_r, vg_r, None, W2_r, b2_r, g_r, bt_r, o_r, st_r))
    in_specs = [
        pl.BlockSpec((R, F1), lambda p, j: (j, 0)),
        pl.BlockSpec((K, R, Dg), lambda p, j: (0, j, 0)),
    ]
    args = [w, vg3]
    if Wg is not None:
        in_specs.append(pl.BlockSpec((Dg, F1), lambda p, j: (0, 0)))
        args.append(Wg)
    in_specs += [
        pl.BlockSpec((F1, F2), lambda p, j: (0, 0)),
        pl.BlockSpec((1, F2), lambda p, j: (0, 0)),
        pl.BlockSpec((1, F1), lambda p, j: (0, 0)),
        pl.BlockSpec((1, F1), lambda p, j: (0, 0)),
    ]
    args += [W2, b2[None], gamma[None], beta[None]]
    return pl.pallas_call(
        body,
        grid=(2, nblk),
        in_specs=in_specs,
        out_specs=pl.BlockSpec((R, F2), lambda p, j: (j, 0)),
        out_shape=jax.ShapeDtypeStruct((N, F2), jnp.float32),
        scratch_shapes=[pltpu.VMEM((8, 2 * F1), jnp.float32)],
        compiler_params=pltpu.CompilerParams(
            vmem_limit_bytes=100 * 1024 * 1024),
    )(*args)


def _edge_conv_mimic(parts, dims, nbr, layers):
    (W1, b1, gamma, beta), (W2, b2) = layers
    np_ = len(parts)
    F1 = W2.shape[0]
    d = sum(dims)
    idx = jnp.reshape(jnp.transpose(nbr[:, :K, :], (1, 0, 2)),
                      (32, E // (32 * 128), 128))
    tables = [_pad_cols(pt, 16) for pt in parts]
    xgs = _sc_gather_multi(tables, idx)
    xg3 = [jnp.reshape(xg, (K, N, t.shape[1]))
           for xg, t in zip(xgs, tables)]
    Wts = _row_splits(W1[:d], dims)
    Wbs = _row_splits(W1[d:], dims)
    nblk = 16
    R = N // nblk

    def body(*refs):
        xi = refs[:np_]
        xg = refs[np_:2 * np_]
        wt = refs[2 * np_:3 * np_]
        wb = refs[3 * np_:4 * np_]
        b1_r, W2_r, b2_r, g_r, bt_r = refs[4 * np_:4 * np_ + 5]
        o_ref, st_ref = refs[-2:]
        p = pl.program_id(0)
        j = pl.program_id(1)
        kk = K
        f1 = F1

        u = None
        for t in range(np_):
            dp = xi[t].shape[1]
            ut = jnp.dot(xi[t][...], wt[t][...],
                         preferred_element_type=jnp.float32)
            u = ut if u is None else u + ut
            diff = xg[t][...][:, :, 0:dp] - xi[t][...][None]
            vt = jnp.dot(jnp.reshape(diff, (kk * R, dp)), wb[t][...],
                         preferred_element_type=jnp.float32)
            vt = jnp.reshape(vt, (kk, R, f1))
            u2 = vt if t == 0 else u2 + vt
        h1 = u[None] + u2 + b1_r[...][None]

        @pl.when((p == 0) & (j == 0))
        def _():
            st_ref[...] = jnp.zeros_like(st_ref)

        @pl.when(p == 0)
        def _():
            s1 = jnp.sum(h1, axis=0)
            s2 = jnp.sum(h1 * h1, axis=0)
            s1 = jnp.sum(jnp.reshape(s1, (R // 8, 8, f1)), axis=0)
            s2 = jnp.sum(jnp.reshape(s2, (R // 8, 8, f1)), axis=0)
            st_ref[...] += jnp.concatenate([s1, s2], axis=1)
            o_ref[...] = jnp.zeros_like(o_ref)

        @pl.when(p == 1)
        def _():
            cnt = float(E)
            tot = jnp.sum(st_ref[...], axis=0, keepdims=True)
            mu = tot[:, 0:f1] / cnt
            var = tot[:, f1:] / cnt - mu * mu
            sq = jnp.sqrt(var + 1e-5)
            hn = (h1 - mu[None]) / sq[None] * g_r[...][None] \
                + bt_r[...][None]
            hr = jnp.maximum(hn, 0.0)
            h2 = jnp.dot(jnp.reshape(hr, (kk * R, f1)), W2_r[...],
                         preferred_element_type=jnp.float32) + b2_r[...]
            f2 = h2.shape[1]
            o_ref[...] = jnp.max(jnp.reshape(h2, (kk, R, f2)), axis=0)

    cst = lambda shape: pl.BlockSpec(shape, lambda p, j: tuple(0 for _ in shape))
    in_specs = (
        [pl.BlockSpec((R, pt.shape[1]), (lambda p, j: (j, 0)))
         for pt in parts]
        + [pl.BlockSpec((K, R, x3.shape[2]), (lambda p, j: (0, j, 0)))
           for x3 in xg3]
        + [cst(wv.shape) for wv in Wts]
        + [cst(wv.shape) for wv in Wbs]
        + [cst((1, F1)), cst((F1, W2.shape[1])), cst((1, W2.shape[1])),
           cst((1, F1)), cst((1, F1))]
    )
    return pl.pallas_call(
        body,
        grid=(2, nblk),
        in_specs=in_specs,
        out_specs=pl.BlockSpec((R, W2.shape[1]), lambda p, j: (j, 0)),
        out_shape=jax.ShapeDtypeStruct((N, W2.shape[1]), jnp.float32),
        scratch_shapes=[pltpu.VMEM((8, 2 * F1), jnp.float32)],
        compiler_params=pltpu.CompilerParams(
            vmem_limit_bytes=100 * 1024 * 1024),
    )(*parts, *xg3, *Wts, *Wbs, b1[None], W2, b2[None], gamma[None],
      beta[None])


def _edge_conv(parts, dims, nbr, layers, raw_gather=False):
    (W1, b1, gamma, beta), (W2, b2) = layers
    d = sum(dims)
    Wt, Wb = W1[:d], W1[d:]
    F1 = W2.shape[0]
    idx = jnp.reshape(jnp.transpose(nbr[:, :K, :], (1, 0, 2)),
                      (32, E // (32 * 128), 128))
    if raw_gather:
        w = _matmul_parts(parts, _row_splits(Wt - Wb, dims), b1[None])
        table = _pad_cols(parts[0], 16) if len(parts) == 1 else None
        dg = table.shape[1]
        vg = _sc_gather(table, idx, dg)
        vg3 = jnp.reshape(vg, (K, N, dg))
        return _edge_stage(w, vg3, W2, b2, gamma, beta,
                           Wg=_pad_rows(Wb, 16))
    Wd = jnp.concatenate([Wt - Wb, Wb], axis=1)
    bias = jnp.concatenate([b1, jnp.zeros_like(b1)])[None]
    uv = _matmul_parts(parts, _row_splits(Wd, dims), bias)
    w = uv[:, :F1]
    v = uv[:, F1:]
    vg = _sc_gather(v, idx, F1)
    vg3 = jnp.reshape(vg, (K, N, F1))
    return _edge_stage(w, vg3, W2, b2, gamma, beta)


def _fe_head(parts, dims, fe, head):
    (W1, b1, gamma, beta), (W2, b2) = fe
    (W0, b0), (W1h, b1h), (W2h, b2h), (W3, b3) = head
    H = W1.shape[1]
    d = sum(dims)
    np_ = len(parts)
    W1s = _row_splits(W1, dims)
    W0ts = _row_splits(W0[:d], dims)
    W0b = W0[d:]
    W3p = _pad_cols(W3, 64)
    b3p = _pad_cols(b3[None], 64)
    nblk = 8
    R = N // nblk

    def body(*refs):
        (o_ref, st_ref, mx_ref) = refs[-3:]
        prefs = refs[:np_]
        w1refs = refs[np_:2 * np_]
        (b1_r, g_r, bt_r, W2_r, b2_r) = refs[2 * np_:2 * np_ + 5]
        w0refs = refs[2 * np_ + 5:3 * np_ + 5]
        (W0b_r, b0_r, W1h_r, b1h_r, W2h_r, b2h_r, W3_r, b3_r) = \
            refs[3 * np_ + 5:3 * np_ + 13]
        p = pl.program_id(0)
        j = pl.program_id(1)

        def hidden():
            t = jnp.dot(prefs[0][...], w1refs[0][...],
                        preferred_element_type=jnp.float32)
            for i in range(1, np_):
                t += jnp.dot(prefs[i][...], w1refs[i][...],
                             preferred_element_type=jnp.float32)
            return t + b1_r[...]

        @pl.when((p == 0) & (j == 0))
        def _():
            st_ref[...] = jnp.zeros_like(st_ref)

        @pl.when(p == 0)
        def _():
            t = hidden()
            s1 = jnp.sum(jnp.reshape(t, (R // 8, 8, H)), axis=0)
            s2 = jnp.sum(jnp.reshape(t * t, (R // 8, 8, H)), axis=0)
            st_ref[...] += jnp.concatenate([s1, s2], axis=1)
            o_ref[...] = jnp.zeros_like(o_ref)

        @pl.when(p == 1)
        def _():
            t = hidden()
            cnt = float(N)
            tot = jnp.sum(st_ref[...], axis=0, keepdims=True)
            mu = tot[:, 0:H] / cnt
            var = tot[:, H:] / cnt - mu * mu
            sc = g_r[...] * (1.0 / jnp.sqrt(var + 1e-5))
            hr = jnp.maximum((t - mu) * sc + bt_r[...], 0.0)
            x5 = jnp.dot(hr, W2_r[...],
                         preferred_element_type=jnp.float32) + b2_r[...]
            bm = jnp.max(x5, axis=0)[None]

            @pl.when(j == 0)
            def _():
                mx_ref[...] = jnp.full_like(mx_ref, -BIG)

            mx_ref[...] = jnp.maximum(mx_ref[...], bm)
            o_ref[...] = jnp.zeros_like(o_ref)

        @pl.when(p == 2)
        def _():
            gbias = jnp.dot(mx_ref[...], W0b_r[...],
                            preferred_element_type=jnp.float32) + b0_r[...]
            h = jnp.dot(prefs[0][...], w0refs[0][...],
                        preferred_element_type=jnp.float32)
            for i in range(1, np_):
                h += jnp.dot(prefs[i][...], w0refs[i][...],
                             preferred_element_type=jnp.float32)
            h = jnp.maximum(h + gbias, 0.0)
            h = jnp.maximum(
                jnp.dot(h, W1h_r[...], preferred_element_type=jnp.float32)
                + b1h_r[...], 0.0)
            h = jnp.maximum(
                jnp.dot(h, W2h_r[...], preferred_element_type=jnp.float32)
                + b2h_r[...], 0.0)
            o = jnp.dot(h, W3_r[...], preferred_element_type=jnp.float32) \
                + b3_r[...]
            colmask = lax.broadcasted_iota(jnp.int32, o.shape, 1) < 50
            o = jnp.where(colmask, o, -BIG)
            m = jnp.max(o, axis=1, keepdims=True)
            z = jnp.sum(jnp.exp(o - m), axis=1, keepdims=True)
            o_ref[...] = o - m - jnp.log(z)

    cst = lambda shape: pl.BlockSpec(shape, lambda p, j: tuple(0 for _ in shape))
    in_specs = (
        [pl.BlockSpec((R, pt.shape[1]), (lambda p, j: (j, 0))) for pt in parts]
        + [cst(wv.shape) for wv in W1s]
        + [cst((1, H)), cst((1, H)), cst((1, H)), cst((H, H)), cst((1, H))]
        + [cst(wv.shape) for wv in W0ts]
        + [cst((H, 256)), cst((1, 256)), cst((256, 256)), cst((1, 256)),
           cst((256, 128)), cst((1, 128)), cst((128, 64)), cst((1, 64))]
    )
    out = pl.pallas_call(
        body,
        grid=(3, nblk),
        in_specs=in_specs,
        out_specs=pl.BlockSpec((R, 64), lambda p, j: (j, 0)),
        out_shape=jax.ShapeDtypeStruct((N, 64), jnp.float32),
        scratch_shapes=[
            pltpu.VMEM((8, 2 * H), jnp.float32),
            pltpu.VMEM((1, H), jnp.float32),
        ],
    )(*parts, *W1s, b1[None], gamma[None], beta[None], W2, b2[None],
      *W0ts, W0b, b0[None], W1h, b1h[None], W2h, b2h[None], W3p, b3p)
    return out[:, :50]


def kernel(x, pos, batch, params):
    x0p = _pad_cols(jnp.concatenate([x, pos], axis=1))

    nbr = _knn(x0p.reshape(B, P, 8))
    x1 = _edge_conv_mimic([x0p], [6], nbr, params["ec1"])

    nbr = _knn(x1.reshape(B, P, 64))
    x2 = _edge_conv_mimic([x0p, x1], [6, 64], nbr, params["ec2"])

    nbr = _knn(x2.reshape(B, P, 64))
    x3 = _edge_conv([x0p, x1, x2], [6, 64, 64], nbr, params["ec3"])

    x4 = _edge_conv([x0p, x1, x2, x3], [6, 64, 64, 64], nbr, params["ec4"])

    return _fe_head([x0p, x1, x2, x3, x4], [6, 64, 64, 64, 128],
                    params["fe"], params["head"])
